# baseline (speedup 1.0000x reference)
import jax
import jax.numpy as jnp
from jax import lax
from jax.experimental import pallas as pl

_H = 32
_W = 32
_C = 192
_P = _H + 2
_PF = _P * _P
_NV = _P * (_H - 1) + _W


def _padded_flat(x):
    xp = jnp.pad(x[0], ((0, 0), (1, 1), (1, 1)), mode="edge")
    return xp.reshape(_C, _PF)


def _patch_match_kernel(s_ref, t_ref, out_ref):
    s = s_ref[:]
    t = t_ref[:]
    g = lax.dot_general(
        s, t,
        dimension_numbers=(((0,), (0,)), ((), ())),
        preferred_element_type=jnp.float32,
    )
    ssq = jnp.sum(s * s, axis=0)
    tsq = jnp.sum(t * t, axis=0)
    cross = jnp.zeros((_NV, _NV), dtype=jnp.float32)
    qsq = jnp.zeros((_NV,), dtype=jnp.float32)
    psq = jnp.zeros((_NV,), dtype=jnp.float32)
    for i in range(3):
        for j in range(3):
            o = _P * i + j
            cross = cross + g[o:o + _NV, o:o + _NV]
            qsq = qsq + ssq[o:o + _NV]
            psq = psq + tsq[o:o + _NV]
    d2 = qsq[:, None] - 2.0 * cross + psq[None, :]
    col = lax.broadcasted_iota(jnp.int32, (_NV, _NV), 1)
    d2 = jnp.where(col % _P < _W, d2, jnp.inf)
    v = jnp.argmin(d2, axis=1).astype(jnp.int32)
    idy = v // _P
    idx = v % _P
    for qh in range(_H):
        u = _P * qh
        out_ref[0, 0, qh, :] = idy[u:u + _W]
        out_ref[0, 1, qh, :] = idx[u:u + _W]


def kernel(s, t):
    n = s.shape[0]
    return pl.pallas_call(
        _patch_match_kernel,
        out_shape=jax.ShapeDtypeStruct((n, 2, _H, _W), jnp.int32),
    )(_padded_flat(s), _padded_flat(t))


# speedup vs baseline: 1.3237x; 1.3237x over previous
import jax
import jax.numpy as jnp
from jax import lax
from jax.experimental import pallas as pl

_H = 32
_W = 32
_C = 192
_P = _H + 2
_PF = _P * _P
_NV = _P * (_H - 1) + _W
_NA = 1152
_PA = 1280


def _padded_flat_t(x):
    xp = jnp.pad(x[0], ((0, 0), (1, 1), (1, 1)), mode="edge")
    return jnp.pad(xp.reshape(_C, _PF), ((0, 0), (0, _PA - _PF))).T


def _patch_match_kernel(st_ref, tt_ref, out_ref):
    st = st_ref[:]
    tt = tt_ref[:]
    ssq = jnp.sum(st * st, axis=1)
    tsq = jnp.sum(tt * tt, axis=1)
    cross = jnp.zeros((_NA, _NA), dtype=jnp.float32)
    qsq = jnp.zeros((_NA,), dtype=jnp.float32)
    psq = jnp.zeros((_NA,), dtype=jnp.float32)
    for i in range(3):
        for j in range(3):
            o = _P * i + j
            cross = cross + lax.dot_general(
                st[o:o + _NA, :], tt[o:o + _NA, :],
                dimension_numbers=(((1,), (1,)), ((), ())),
                preferred_element_type=jnp.float32,
            )
            qsq = qsq + ssq[o:o + _NA]
            psq = psq + tsq[o:o + _NA]
    d2 = qsq[:, None] - 2.0 * cross + psq[None, :]
    col = lax.broadcasted_iota(jnp.int32, (_NA, _NA), 1)
    d2 = jnp.where((col % _P < _W) & (col < _NV), d2, jnp.inf)
    v = jnp.argmin(d2, axis=1).astype(jnp.int32)
    idy = v // _P
    idx = v % _P
    for qh in range(_H):
        u = _P * qh
        out_ref[0, 0, qh, :] = idy[u:u + _W]
        out_ref[0, 1, qh, :] = idx[u:u + _W]


def kernel(s, t):
    n = s.shape[0]
    return pl.pallas_call(
        _patch_match_kernel,
        out_shape=jax.ShapeDtypeStruct((n, 2, _H, _W), jnp.int32),
    )(_padded_flat_t(s), _padded_flat_t(t))
